# baseline (speedup 1.0000x reference)
import jax
import jax.numpy as jnp
from jax.experimental import pallas as pl
from jax.experimental.pallas import tpu as pltpu

_B = 1024
_S = 4096
_K = 4
_E = 16
_SPG = 32
_G = _S // _SPG
_LIN = _SPG * _K
_LOUT = _SPG * _E
_BM = 8


def _body(x_ref, w_ref, p_ref, o_ref):
    y = jnp.dot(x_ref[...], w_ref[...], preferred_element_type=jnp.float32)
    y = y.reshape(_BM, _G, _LOUT) + p_ref[...][None]
    o_ref[...] = y.reshape(_BM * _G, _LOUT)


def kernel(inputs, embedding, pos_table):
    x = inputs.reshape(_B * _G, _LIN)
    p = pos_table.reshape(_G, _LOUT)
    i = jax.lax.broadcasted_iota(jnp.int32, (_LIN, _LOUT), 0)
    j = jax.lax.broadcasted_iota(jnp.int32, (_LIN, _LOUT), 1)
    w = jnp.where(i // _K == j // _E, jnp.tile(embedding, (_SPG, _SPG)), 0.0)

    rows = _BM * _G
    out = pl.pallas_call(
        _body,
        grid=(_B // _BM,),
        in_specs=[
            pl.BlockSpec((rows, _LIN), lambda m: (m, 0)),
            pl.BlockSpec((_LIN, _LOUT), lambda m: (0, 0)),
            pl.BlockSpec((_G, _LOUT), lambda m: (0, 0)),
        ],
        out_specs=pl.BlockSpec((rows, _LOUT), lambda m: (m, 0)),
        out_shape=jax.ShapeDtypeStruct((_B * _G, _LOUT), jnp.float32),
        compiler_params=pltpu.CompilerParams(
            dimension_semantics=("parallel",),
        ),
    )(x, w, p)
    return out.reshape(_B, _S, _E)


# speedup vs baseline: 37.3206x; 37.3206x over previous
import jax
import jax.numpy as jnp
from jax.experimental import pallas as pl
from jax.experimental.pallas import tpu as pltpu

_B = 1024
_S = 4096
_K = 4
_E = 16
_BM = 8
_BS = 4096


def _body(x_ref, e_ref, p_ref, o_ref):
    acc = jnp.broadcast_to(p_ref[...][None], (_BM, _E, _BS))
    for k in range(_K):
        xk = x_ref[:, k, :][:, None, :]
        ek = e_ref[:, k][None, :, None]
        acc = acc + xk * ek
    o_ref[...] = acc


def kernel(inputs, embedding, pos_table):
    xt = jnp.transpose(inputs, (0, 2, 1))
    pt = jnp.transpose(pos_table, (1, 0))
    et = jnp.transpose(embedding, (1, 0))
    out_t = pl.pallas_call(
        _body,
        grid=(_B // _BM, _S // _BS),
        in_specs=[
            pl.BlockSpec((_BM, _K, _BS), lambda i, j: (i, 0, j)),
            pl.BlockSpec((_E, _K), lambda i, j: (0, 0)),
            pl.BlockSpec((_E, _BS), lambda i, j: (0, j)),
        ],
        out_specs=pl.BlockSpec((_BM, _E, _BS), lambda i, j: (i, 0, j)),
        out_shape=jax.ShapeDtypeStruct((_B, _E, _S), jnp.float32),
        compiler_params=pltpu.CompilerParams(
            dimension_semantics=("parallel", "parallel"),
        ),
    )(xt, et, pt)
    return jnp.transpose(out_t, (0, 2, 1))


# speedup vs baseline: 47.9898x; 1.2859x over previous
import jax
import jax.numpy as jnp
from jax.experimental import pallas as pl
from jax.experimental.pallas import tpu as pltpu

_B = 1024
_S = 4096
_K = 4
_E = 16
_BM = 8
_BS = 4096


def _body(x_ref, e_ref, p_ref, o_ref):
    p = p_ref[...]
    e = e_ref[...]
    for b in range(_BM):
        y = jnp.dot(e, x_ref[b], preferred_element_type=jnp.float32)
        o_ref[b] = y + p


def kernel(inputs, embedding, pos_table):
    xt = jnp.transpose(inputs, (0, 2, 1))
    pt = jnp.transpose(pos_table, (1, 0))
    et = jnp.transpose(embedding, (1, 0))
    out_t = pl.pallas_call(
        _body,
        grid=(_B // _BM, _S // _BS),
        in_specs=[
            pl.BlockSpec((_BM, _K, _BS), lambda i, j: (i, 0, j)),
            pl.BlockSpec((_E, _K), lambda i, j: (0, 0)),
            pl.BlockSpec((_E, _BS), lambda i, j: (0, j)),
        ],
        out_specs=pl.BlockSpec((_BM, _E, _BS), lambda i, j: (i, 0, j)),
        out_shape=jax.ShapeDtypeStruct((_B, _E, _S), jnp.float32),
        compiler_params=pltpu.CompilerParams(
            dimension_semantics=("parallel", "parallel"),
        ),
    )(xt, et, pt)
    return jnp.transpose(out_t, (0, 2, 1))


# speedup vs baseline: 63.2909x; 1.3188x over previous
import jax
import jax.numpy as jnp
from jax.experimental import pallas as pl
from jax.experimental.pallas import tpu as pltpu

_B = 1024
_S = 4096
_K = 4
_E = 16
_BM = 16
_BS = 4096


def _body(x_ref, e_ref, p_ref, o_ref):
    p = p_ref[...]
    e = e_ref[...]
    for b in range(_BM):
        y = jnp.dot(e, x_ref[b], preferred_element_type=jnp.float32)
        o_ref[b] = y + p


def kernel(inputs, embedding, pos_table):
    xt = jnp.transpose(inputs, (0, 2, 1))
    pt = jnp.transpose(pos_table, (1, 0))
    et = jnp.transpose(embedding, (1, 0))
    out_t = pl.pallas_call(
        _body,
        grid=(_B // _BM, _S // _BS),
        in_specs=[
            pl.BlockSpec((_BM, _K, _BS), lambda i, j: (i, 0, j)),
            pl.BlockSpec((_E, _K), lambda i, j: (0, 0)),
            pl.BlockSpec((_E, _BS), lambda i, j: (0, j)),
        ],
        out_specs=pl.BlockSpec((_BM, _E, _BS), lambda i, j: (i, 0, j)),
        out_shape=jax.ShapeDtypeStruct((_B, _E, _S), jnp.float32),
        compiler_params=pltpu.CompilerParams(
            dimension_semantics=("parallel", "parallel"),
        ),
    )(xt, et, pt)
    return jnp.transpose(out_t, (0, 2, 1))


# speedup vs baseline: 69.1882x; 1.0932x over previous
import jax
import jax.numpy as jnp
from jax.experimental import pallas as pl
from jax.experimental.pallas import tpu as pltpu

_B = 1024
_S = 4096
_K = 4
_E = 16
_BM = 32
_BS = 4096


def _body(x_ref, e_ref, p_ref, o_ref):
    p = p_ref[...]
    e = e_ref[...]
    for b in range(_BM):
        y = jnp.dot(e, x_ref[b], preferred_element_type=jnp.float32)
        o_ref[b] = y + p


def kernel(inputs, embedding, pos_table):
    xt = jnp.transpose(inputs, (0, 2, 1))
    pt = jnp.transpose(pos_table, (1, 0))
    et = jnp.transpose(embedding, (1, 0))
    out_t = pl.pallas_call(
        _body,
        grid=(_B // _BM, _S // _BS),
        in_specs=[
            pl.BlockSpec((_BM, _K, _BS), lambda i, j: (i, 0, j)),
            pl.BlockSpec((_E, _K), lambda i, j: (0, 0)),
            pl.BlockSpec((_E, _BS), lambda i, j: (0, j)),
        ],
        out_specs=pl.BlockSpec((_BM, _E, _BS), lambda i, j: (i, 0, j)),
        out_shape=jax.ShapeDtypeStruct((_B, _E, _S), jnp.float32),
        compiler_params=pltpu.CompilerParams(
            dimension_semantics=("parallel", "parallel"),
        ),
    )(xt, et, pt)
    return jnp.transpose(out_t, (0, 2, 1))


# speedup vs baseline: 70.7094x; 1.0220x over previous
import jax
import jax.numpy as jnp
from jax.experimental import pallas as pl
from jax.experimental.pallas import tpu as pltpu

_B = 1024
_S = 4096
_K = 4
_E = 16
_BM = 64
_BS = 4096


def _body(x_ref, e_ref, p_ref, o_ref):
    p = p_ref[...]
    e = e_ref[...]
    for b in range(_BM):
        y = jnp.dot(e, x_ref[b], preferred_element_type=jnp.float32)
        o_ref[b] = y + p


def kernel(inputs, embedding, pos_table):
    xt = jnp.transpose(inputs, (0, 2, 1))
    pt = jnp.transpose(pos_table, (1, 0))
    et = jnp.transpose(embedding, (1, 0))
    out_t = pl.pallas_call(
        _body,
        grid=(_B // _BM, _S // _BS),
        in_specs=[
            pl.BlockSpec((_BM, _K, _BS), lambda i, j: (i, 0, j)),
            pl.BlockSpec((_E, _K), lambda i, j: (0, 0)),
            pl.BlockSpec((_E, _BS), lambda i, j: (0, j)),
        ],
        out_specs=pl.BlockSpec((_BM, _E, _BS), lambda i, j: (i, 0, j)),
        out_shape=jax.ShapeDtypeStruct((_B, _E, _S), jnp.float32),
        compiler_params=pltpu.CompilerParams(
            dimension_semantics=("parallel", "parallel"),
        ),
    )(xt, et, pt)
    return jnp.transpose(out_t, (0, 2, 1))
